# baseline (speedup 1.0000x reference)
import functools

import numpy as np

import jax
import jax.numpy as jnp
from jax import lax
from jax.experimental import pallas as pl
from jax.experimental.pallas import tpu as pltpu
from jax.experimental.pallas import tpu_sc as plsc

GRID_SIZE = 3
SPACING = 2
IMAGE_SIZE = 512
GSD = 0.2
C = 5
D = 256
TOKEN_DIM = 6

B = 2
L = 512
S = GRID_SIZE * GRID_SIZE
N = L * S
Q = N * C
ROWS = B * N * C
WORDS = ROWS * TOKEN_DIM
NW = 32
WPW = WORDS // NW

BL = 128
NJ = L // BL
CL = 32
NCH = BL // CL
GRP = S * C
CQ = CL * GRP

HW = IMAGE_SIZE * IMAGE_SIZE
TROWS = B * C * TOKEN_DIM * HW // 8


_P = np.arange(240)
_BNOFF = (_P // (C * TOKEN_DIM)).astype(np.int32)
_SOFF = ((_P % (C * TOKEN_DIM)) * HW).astype(np.int32)
_SG_BN = 8
_NSG = WPW // 240


def _sc_gather(table_flat, within2, bnoff, soff):
    mesh = plsc.VectorSubcoreMesh(core_axis_name="c", subcore_axis_name="s")

    @functools.partial(
        pl.kernel,
        out_type=jax.ShapeDtypeStruct((WORDS,), jnp.float32),
        mesh=mesh,
        scratch_types=[
            pltpu.VMEM((B * N + 16,), jnp.int32),
            pltpu.VMEM((240,), jnp.int32),
            pltpu.VMEM((240,), jnp.int32),
            pltpu.VMEM((WPW,), jnp.int32),
            pltpu.VMEM((WPW,), jnp.float32),
            pltpu.SemaphoreType.DMA,
        ],
        compiler_params=pltpu.CompilerParams(use_tc_tiling_on_sc=False),
    )
    def k(table_hbm, win_hbm, bnoff_hbm, soff_hbm, out_hbm, win_v, bnoff_v,
          soff_v, addr_v, out_v, sem):
        wid = lax.axis_index("s") * 2 + lax.axis_index("c")
        base = wid * WPW
        base_bn = wid * (WPW // (C * TOKEN_DIM))
        pltpu.sync_copy(win_hbm, win_v.at[pl.ds(0, B * N)])
        pltpu.sync_copy(bnoff_hbm, bnoff_v)
        pltpu.sync_copy(soff_hbm, soff_v)

        dnums = lax.GatherDimensionNumbers(
            offset_dims=(), collapsed_slice_dims=(0,), start_index_map=(0,))

        def body(g, _):
            bn0 = base_bn + g * _SG_BN
            win16 = win_v[pl.ds(bn0, 16)]
            for j in range(15):
                win = lax.gather(
                    win16, bnoff_v[pl.ds(j * 16, 16)][:, None], dnums, (1,),
                    mode=lax.GatherScatterMode.PROMISE_IN_BOUNDS)
                addr_v[pl.ds(g * 240 + j * 16, 16)] = (
                    win + soff_v[pl.ds(j * 16, 16)])
            return 0

        lax.fori_loop(0, _NSG, body, 0)
        pltpu.async_copy(table_hbm.at[addr_v], out_v, sem).wait()
        pltpu.sync_copy(out_v, out_hbm.at[pl.ds(base, WPW)])

    return k(table_flat, within2, bnoff, soff)


def _tc_body(tokens_ref, latents_ref, coords_ref, wq_ref, wk_ref, wpk_ref,
             wv_ref, wout_ref, bout_ref, out_ref, pt_scr, vw_scr):
    j = pl.program_id(1)

    @pl.when(j == 0)
    def _():
        lat = latents_ref[0]
        kmat = (jnp.dot(lat, wk_ref[...], preferred_element_type=jnp.float32)
                + jnp.dot(coords_ref[0], wpk_ref[...],
                          preferred_element_type=jnp.float32))
        pt_scr[...] = lax.dot_general(
            wq_ref[...], kmat, (((1,), (1,)), ((), ())),
            preferred_element_type=jnp.float32) * (1.0 / (D ** 0.5))
        u = jnp.dot(wv_ref[...], wout_ref[...],
                    preferred_element_type=jnp.float32)
        vw_scr[...] = lax.dot_general(
            u, lat, (((0,), (1,)), ((), ())),
            preferred_element_type=jnp.float32)

    p_mat = pt_scr[...]
    vw2 = jnp.concatenate(
        [vw_scr[...], jnp.ones((1, L), jnp.float32)], axis=0)
    b_out = bout_ref[0, 0]
    errs = []
    for ch in range(NCH):
        toks6 = tokens_ref[pl.ds(ch * CQ, CQ), :]
        toks = jnp.concatenate(
            [toks6, jnp.zeros((CQ, 2), jnp.float32)], axis=1)
        scores = lax.dot_general(
            toks, p_mat, (((1,), (0,)), ((), ())),
            preferred_element_type=jnp.float32)
        m = jnp.max(scores, axis=1, keepdims=True)
        e = jnp.exp(scores - m)
        wz = lax.dot_general(e, vw2, (((1,), (1,)), ((), ())),
                             preferred_element_type=jnp.float32)
        pred = wz[:, 0:1] / wz[:, 1:2] + b_out
        sq = (pred - toks6[:, 0:1]) ** 2
        rows_l = lax.broadcasted_iota(jnp.int32, (CQ, CL), 0) // GRP
        cols_l = lax.broadcasted_iota(jnp.int32, (CQ, CL), 1)
        msk = (rows_l == cols_l).astype(jnp.float32)
        err = lax.dot_general(sq, msk, (((0,), (0,)), ((), ())),
                              preferred_element_type=jnp.float32)
        errs.append(err * (1.0 / GRP))
    out_ref[...] = jnp.concatenate(errs, axis=1).reshape(1, 1, 1, BL)


def _tc_main(tokens, latents, latent_coords, Wq8, Wk, Wpk, Wv, w_out2,
             b_out2):
    tq = BL * GRP
    out = pl.pallas_call(
        _tc_body,
        grid=(B, NJ),
        in_specs=[
            pl.BlockSpec((tq, TOKEN_DIM), lambda b, j: (b * NJ + j, 0)),
            pl.BlockSpec((1, L, D), lambda b, j: (b, 0, 0)),
            pl.BlockSpec((1, L, 2), lambda b, j: (b, 0, 0)),
            pl.BlockSpec((8, D), lambda b, j: (0, 0)),
            pl.BlockSpec((D, D), lambda b, j: (0, 0)),
            pl.BlockSpec((2, D), lambda b, j: (0, 0)),
            pl.BlockSpec((D, D), lambda b, j: (0, 0)),
            pl.BlockSpec((D, 1), lambda b, j: (0, 0)),
            pl.BlockSpec((1, 1), lambda b, j: (0, 0)),
        ],
        out_specs=pl.BlockSpec((1, 1, 1, BL), lambda b, j: (b, j, 0, 0)),
        out_shape=jax.ShapeDtypeStruct((B, NJ, 1, BL), jnp.float32),
        scratch_shapes=[
            pltpu.VMEM((8, L), jnp.float32),
            pltpu.VMEM((1, L), jnp.float32),
        ],
    )(tokens, latents, latent_coords, Wq8, Wk, Wpk, Wv, w_out2, b_out2)
    return out.reshape(B, L)


def kernel(positions, latents, latent_coords, image_err, Wq, Wk, Wpk, Wv,
           w_out, b_out):
    pix = positions / GSD + IMAGE_SIZE / 2.0
    offs = (jnp.arange(GRID_SIZE, dtype=jnp.float32)
            - (GRID_SIZE - 1) / 2.0) * SPACING
    oy, ox = jnp.meshgrid(offs, offs, indexing="ij")
    grid = jnp.stack([ox.ravel(), oy.ravel()], axis=-1)
    coords = pix[:, :, None, :] + grid[None, None, :, :]
    coords = jnp.clip(coords, 0.0, IMAGE_SIZE - 1.0).reshape(B, N, 2)
    idx = jnp.round(coords).astype(jnp.int32)
    x = jnp.clip(idx[..., 0], 0, IMAGE_SIZE - 1)
    y = jnp.clip(idx[..., 1], 0, IMAGE_SIZE - 1)

    within = ((y >> 3) * 4096 + (x >> 7) * 1024
              + (y & 7) * 128 + (x & 127))
    boff = (jnp.arange(B, dtype=jnp.int32)
            * (C * TOKEN_DIM * HW))[:, None]
    within2 = (within + boff).reshape(B * N)

    t1 = jnp.transpose(image_err, (0, 1, 4, 2, 3))
    t2 = t1.reshape(B, C, TOKEN_DIM, IMAGE_SIZE // 8, 8,
                    IMAGE_SIZE // 128, 128)
    t3 = jnp.transpose(t2, (0, 1, 2, 3, 5, 4, 6))
    table_flat = t3.reshape(TROWS * 8)

    vals = _sc_gather(table_flat, within2, jnp.asarray(_BNOFF),
                      jnp.asarray(_SOFF)).reshape(B * Q, TOKEN_DIM)

    Wq8 = jnp.pad(Wq, ((0, 2), (0, 0)))
    return _tc_main(vals, latents, latent_coords, Wq8, Wk, Wpk, Wv,
                    w_out.reshape(D, 1), b_out.reshape(1, 1))


# speedup vs baseline: 1.1180x; 1.1180x over previous
import functools

import numpy as np

import jax
import jax.numpy as jnp
from jax import lax
from jax.experimental import pallas as pl
from jax.experimental.pallas import tpu as pltpu
from jax.experimental.pallas import tpu_sc as plsc

GRID_SIZE = 3
SPACING = 2
IMAGE_SIZE = 512
GSD = 0.2
C = 5
D = 256
TOKEN_DIM = 6

B = 2
L = 512
S = GRID_SIZE * GRID_SIZE
N = L * S
Q = N * C
ROWS = B * N * C
WORDS = ROWS * TOKEN_DIM
NW = 32
WPW = WORDS // NW

BL = 128
NJ = L // BL
CL = 32
NCH = BL // CL
GRP = S * C
CQ = CL * GRP

HW = IMAGE_SIZE * IMAGE_SIZE
TROWS = B * C * TOKEN_DIM * HW // 8


_P = np.arange(240)
_BNOFF = (_P // (C * TOKEN_DIM)).astype(np.int32)
_SOFF = ((_P % (C * TOKEN_DIM)) * HW).astype(np.int32)
_SG_BN = 8
_NSG = WPW // 240


def _sc_gather(table_flat, within2, bnoff, soff):
    mesh = plsc.VectorSubcoreMesh(core_axis_name="c", subcore_axis_name="s")

    @functools.partial(
        pl.kernel,
        out_type=jax.ShapeDtypeStruct((WORDS,), jnp.float32),
        mesh=mesh,
        scratch_types=[
            pltpu.VMEM((B * N + 16,), jnp.int32),
            pltpu.VMEM((240,), jnp.int32),
            pltpu.VMEM((240,), jnp.int32),
            pltpu.VMEM((WPW,), jnp.int32),
            pltpu.VMEM((WPW,), jnp.float32),
            pltpu.SemaphoreType.DMA,
        ],
        compiler_params=pltpu.CompilerParams(use_tc_tiling_on_sc=False),
    )
    def k(table_hbm, win_hbm, bnoff_hbm, soff_hbm, out_hbm, win_v, bnoff_v,
          soff_v, addr_v, out_v, sem):
        wid = lax.axis_index("s") * 2 + lax.axis_index("c")
        base = wid * WPW
        base_bn = wid * (WPW // (C * TOKEN_DIM))
        pltpu.sync_copy(win_hbm, win_v.at[pl.ds(0, B * N)])
        pltpu.sync_copy(bnoff_hbm, bnoff_v)
        pltpu.sync_copy(soff_hbm, soff_v)

        dnums = lax.GatherDimensionNumbers(
            offset_dims=(), collapsed_slice_dims=(0,), start_index_map=(0,))

        def body(g, _):
            bn0 = base_bn + g * _SG_BN
            win16 = win_v[pl.ds(bn0, 16)]
            for j in range(15):
                win = lax.gather(
                    win16, bnoff_v[pl.ds(j * 16, 16)][:, None], dnums, (1,),
                    mode=lax.GatherScatterMode.PROMISE_IN_BOUNDS)
                addr_v[pl.ds(g * 240 + j * 16, 16)] = (
                    win + soff_v[pl.ds(j * 16, 16)])
            return 0

        lax.fori_loop(0, _NSG, body, 0)
        pltpu.async_copy(table_hbm.at[addr_v], out_v, sem).wait()
        pltpu.sync_copy(out_v, out_hbm.at[pl.ds(base, WPW)])

    return k(table_flat, within2, bnoff, soff)


def _tc_body(tokens_ref, latents_ref, coords_ref, wq_ref, wk_ref, wpk_ref,
             wv_ref, wout_ref, bout_ref, out_ref, pt_scr, vw_scr):
    j = pl.program_id(1)

    @pl.when(j == 0)
    def _():
        lat = latents_ref[0]
        kmat = (jnp.dot(lat, wk_ref[...], preferred_element_type=jnp.float32)
                + jnp.dot(coords_ref[0], wpk_ref[...],
                          preferred_element_type=jnp.float32))
        pt_scr[...] = lax.dot_general(
            wq_ref[...], kmat, (((1,), (1,)), ((), ())),
            preferred_element_type=jnp.float32) * (1.0 / (D ** 0.5))
        u = jnp.dot(wv_ref[...], wout_ref[...],
                    preferred_element_type=jnp.float32)
        vw_scr[...] = lax.dot_general(
            u, lat, (((0,), (1,)), ((), ())),
            preferred_element_type=jnp.float32)

    p_mat = pt_scr[...]
    vw = vw_scr[...]
    b_out = bout_ref[0, 0]
    errs = []
    for ch in range(NCH):
        toks6 = tokens_ref[pl.ds(ch * CQ, CQ), :]
        toks = jnp.concatenate(
            [toks6, jnp.zeros((CQ, 2), jnp.float32)], axis=1)
        scores = lax.dot_general(
            toks, p_mat, (((1,), (0,)), ((), ())),
            preferred_element_type=jnp.float32)
        m = jnp.max(scores, axis=1, keepdims=True)
        e = jnp.exp(scores - m)
        z = jnp.sum(e, axis=1, keepdims=True)
        w = lax.dot_general(e, vw, (((1,), (1,)), ((), ())),
                            preferred_element_type=jnp.float32)
        pred = w / z + b_out
        sq = (pred - toks6[:, 0:1]) ** 2
        rows_l = lax.broadcasted_iota(jnp.int32, (CQ, CL), 0) // GRP
        cols_l = lax.broadcasted_iota(jnp.int32, (CQ, CL), 1)
        msk = (rows_l == cols_l).astype(jnp.float32)
        err = lax.dot_general(sq, msk, (((0,), (0,)), ((), ())),
                              preferred_element_type=jnp.float32)
        errs.append(err * (1.0 / GRP))
    out_ref[...] = jnp.concatenate(errs, axis=1).reshape(1, 1, 1, BL)


def _tc_main(tokens, latents, latent_coords, Wq8, Wk, Wpk, Wv, w_out2,
             b_out2):
    tq = BL * GRP
    out = pl.pallas_call(
        _tc_body,
        grid=(B, NJ),
        in_specs=[
            pl.BlockSpec((tq, TOKEN_DIM), lambda b, j: (b * NJ + j, 0)),
            pl.BlockSpec((1, L, D), lambda b, j: (b, 0, 0)),
            pl.BlockSpec((1, L, 2), lambda b, j: (b, 0, 0)),
            pl.BlockSpec((8, D), lambda b, j: (0, 0)),
            pl.BlockSpec((D, D), lambda b, j: (0, 0)),
            pl.BlockSpec((2, D), lambda b, j: (0, 0)),
            pl.BlockSpec((D, D), lambda b, j: (0, 0)),
            pl.BlockSpec((D, 1), lambda b, j: (0, 0)),
            pl.BlockSpec((1, 1), lambda b, j: (0, 0)),
        ],
        out_specs=pl.BlockSpec((1, 1, 1, BL), lambda b, j: (b, j, 0, 0)),
        out_shape=jax.ShapeDtypeStruct((B, NJ, 1, BL), jnp.float32),
        scratch_shapes=[
            pltpu.VMEM((8, L), jnp.float32),
            pltpu.VMEM((1, L), jnp.float32),
        ],
    )(tokens, latents, latent_coords, Wq8, Wk, Wpk, Wv, w_out2, b_out2)
    return out.reshape(B, L)


def kernel(positions, latents, latent_coords, image_err, Wq, Wk, Wpk, Wv,
           w_out, b_out):
    pix = positions / GSD + IMAGE_SIZE / 2.0
    offs = (jnp.arange(GRID_SIZE, dtype=jnp.float32)
            - (GRID_SIZE - 1) / 2.0) * SPACING
    oy, ox = jnp.meshgrid(offs, offs, indexing="ij")
    grid = jnp.stack([ox.ravel(), oy.ravel()], axis=-1)
    coords = pix[:, :, None, :] + grid[None, None, :, :]
    coords = jnp.clip(coords, 0.0, IMAGE_SIZE - 1.0).reshape(B, N, 2)
    idx = jnp.round(coords).astype(jnp.int32)
    x = jnp.clip(idx[..., 0], 0, IMAGE_SIZE - 1)
    y = jnp.clip(idx[..., 1], 0, IMAGE_SIZE - 1)

    within = ((y >> 3) * 4096 + (x >> 7) * 1024
              + (y & 7) * 128 + (x & 127))
    boff = (jnp.arange(B, dtype=jnp.int32)
            * (C * TOKEN_DIM * HW))[:, None]
    within2 = (within + boff).reshape(B * N)

    t1 = jnp.transpose(image_err, (0, 1, 4, 2, 3))
    t2 = t1.reshape(B, C, TOKEN_DIM, IMAGE_SIZE // 8, 8,
                    IMAGE_SIZE // 128, 128)
    t3 = jnp.transpose(t2, (0, 1, 2, 3, 5, 4, 6))
    table_flat = t3.reshape(TROWS * 8)

    vals = _sc_gather(table_flat, within2, jnp.asarray(_BNOFF),
                      jnp.asarray(_SOFF)).reshape(B * Q, TOKEN_DIM)

    Wq8 = jnp.pad(Wq, ((0, 2), (0, 0)))
    return _tc_main(vals, latents, latent_coords, Wq8, Wk, Wpk, Wv,
                    w_out.reshape(D, 1), b_out.reshape(1, 1))
